# Initial kernel scaffold; baseline (speedup 1.0000x reference)
import jax
import jax.numpy as jnp
from jax.experimental import pallas as pl


def kernel(x, edge_index, W1, b1, W2, b2):
    raise NotImplementedError("write your pallas kernel here")


# speedup vs baseline: 15.9528x; 15.9528x over previous
import functools

import jax
import jax.numpy as jnp
from jax import lax
from jax.experimental import pallas as pl
from jax.experimental.pallas import tpu as pltpu
from jax.experimental.pallas import tpu_sc as plsc

NC = 2
NS = 16
_W = NC * NS
_B = 128


def _sc_degree(dst, n_nodes):
  e = dst.shape[0]
  epw = e // _W
  nfull, tail = divmod(epw, _B)
  ra = (n_nodes // NS) // 8 * 8
  res = n_nodes - NS * ra
  mesh = plsc.VectorSubcoreMesh(
      core_axis_name="c", subcore_axis_name="s",
      num_cores=NC, num_subcores=NS)

  @functools.partial(
      pl.kernel,
      out_type=jax.ShapeDtypeStruct((NC, n_nodes, 16), jnp.float32),
      mesh=mesh,
      scratch_types=[
          pltpu.VMEM((_B,), jnp.int32),
          pltpu.VMEM((tail,), jnp.int32),
          pltpu.VMEM((_B, 16), jnp.float32),
          pltpu.VMEM_SHARED((n_nodes, 16), jnp.float32),
      ],
  )
  def deg_kernel(dst_hbm, out_hbm, didx, didx_t, ones_v, cnt_sh):
    cid = lax.axis_index("c")
    sid = lax.axis_index("s")
    wid = sid * NC + cid
    base = wid * epw
    tb = sid * ra

    def fill_row(i, carry):
      ones_v[i, :] = jnp.full((16,), 1.0, jnp.float32)
      return carry
    lax.fori_loop(0, _B, fill_row, 0)

    done = 0
    while done < ra:
      sz = min(_B, ra - done)
      pltpu.sync_copy(ones_v.at[pl.ds(0, sz)],
                      cnt_sh.at[pl.ds(tb + done, sz)])
      done += sz
    if res:
      @pl.when(sid == NS - 1)
      def _():
        pltpu.sync_copy(ones_v.at[pl.ds(0, res)],
                        cnt_sh.at[pl.ds(NS * ra, res)])
    plsc.subcore_barrier()

    def batch(i, carry):
      pltpu.sync_copy(dst_hbm.at[pl.ds(base + i * _B, _B)], didx)
      pltpu.sync_copy(ones_v, cnt_sh.at[didx], add=True)
      return carry
    lax.fori_loop(0, nfull, batch, 0)
    if tail:
      pltpu.sync_copy(dst_hbm.at[pl.ds(base + nfull * _B, tail)], didx_t)
      pltpu.sync_copy(ones_v.at[pl.ds(0, tail)], cnt_sh.at[didx_t], add=True)
    plsc.subcore_barrier()
    pltpu.sync_copy(cnt_sh.at[pl.ds(tb, ra)],
                    out_hbm.at[cid, pl.ds(tb, ra)])
    if res:
      @pl.when(sid == NS - 1)
      def _():
        pltpu.sync_copy(cnt_sh.at[pl.ds(NS * ra, res)],
                        out_hbm.at[cid, pl.ds(NS * ra, res)])

  return deg_kernel(dst)


def _sc_edge_agg(src, dst, hp):
  n_nodes, f = hp.shape
  e = src.shape[0]
  epw = e // _W
  nfull, tail = divmod(epw, _B)
  ra = (n_nodes // NS) // 8 * 8
  res = n_nodes - NS * ra
  mesh = plsc.VectorSubcoreMesh(
      core_axis_name="c", subcore_axis_name="s",
      num_cores=NC, num_subcores=NS)

  @functools.partial(
      pl.kernel,
      out_type=jax.ShapeDtypeStruct((NC, n_nodes, f), jnp.float32),
      mesh=mesh,
      scratch_types=[
          pltpu.VMEM((_B,), jnp.int32),
          pltpu.VMEM((_B,), jnp.int32),
          pltpu.VMEM((tail,), jnp.int32),
          pltpu.VMEM((tail,), jnp.int32),
          pltpu.VMEM((_B, f), jnp.float32),
          pltpu.VMEM_SHARED((n_nodes, f), jnp.float32),
          pltpu.SemaphoreType.DMA,
      ],
  )
  def agg_kernel(src_hbm, dst_hbm, hp_hbm, out_hbm,
                 sidx, didx, sidx_t, didx_t, rows, agg_sh, sem):
    cid = lax.axis_index("c")
    sid = lax.axis_index("s")
    wid = sid * NC + cid
    base = wid * epw
    tb = sid * ra

    pltpu.sync_copy(hp_hbm.at[pl.ds(tb, ra)], agg_sh.at[pl.ds(tb, ra)])
    if res:
      @pl.when(sid == NS - 1)
      def _():
        pltpu.sync_copy(hp_hbm.at[pl.ds(NS * ra, res)],
                        agg_sh.at[pl.ds(NS * ra, res)])
    plsc.subcore_barrier()

    def batch(i, carry):
      off = base + i * _B
      pltpu.sync_copy(src_hbm.at[pl.ds(off, _B)], sidx)
      pltpu.sync_copy(dst_hbm.at[pl.ds(off, _B)], didx)
      pltpu.async_copy(hp_hbm.at[sidx], rows, sem).wait()
      pltpu.sync_copy(rows, agg_sh.at[didx], add=True)
      return carry
    lax.fori_loop(0, nfull, batch, 0)
    if tail:
      off = base + nfull * _B
      pltpu.sync_copy(src_hbm.at[pl.ds(off, tail)], sidx_t)
      pltpu.sync_copy(dst_hbm.at[pl.ds(off, tail)], didx_t)
      pltpu.async_copy(hp_hbm.at[sidx_t], rows.at[pl.ds(0, tail)], sem).wait()
      pltpu.sync_copy(rows.at[pl.ds(0, tail)], agg_sh.at[didx_t], add=True)
    plsc.subcore_barrier()
    pltpu.sync_copy(agg_sh.at[pl.ds(tb, ra)],
                    out_hbm.at[cid, pl.ds(tb, ra)])
    if res:
      @pl.when(sid == NS - 1)
      def _():
        pltpu.sync_copy(agg_sh.at[pl.ds(NS * ra, res)],
                        out_hbm.at[cid, pl.ds(NS * ra, res)])

  return agg_kernel(src, dst, hp)


_TC_PARAMS = pltpu.CompilerParams(
    dimension_semantics=("arbitrary",))


def _tc_first(x, w1, deg_parts, blk):
  n, nf = x.shape
  h = w1.shape[1]

  def body(deg_ref, x_ref, w_ref, dinv_ref, h1p_ref):
    d = deg_ref[0, :, 0:1] + deg_ref[1, :, 0:1] - 1.0
    dinv = lax.rsqrt(d)
    dinv_ref[...] = dinv
    h1p_ref[...] = jnp.dot(x_ref[...], w_ref[...],
                           preferred_element_type=jnp.float32) * dinv

  grid = (n // blk,)
  return pl.pallas_call(
      body,
      grid=grid,
      in_specs=[
          pl.BlockSpec((NC, blk, 16), lambda i: (0, i, 0)),
          pl.BlockSpec((blk, nf), lambda i: (i, 0)),
          pl.BlockSpec((nf, h), lambda i: (0, 0)),
      ],
      out_specs=[
          pl.BlockSpec((blk, 1), lambda i: (i, 0)),
          pl.BlockSpec((blk, h), lambda i: (i, 0)),
      ],
      out_shape=[
          jax.ShapeDtypeStruct((n, 1), jnp.float32),
          jax.ShapeDtypeStruct((n, h), jnp.float32),
      ],
      compiler_params=_TC_PARAMS,
  )(deg_parts, x, w1)


def _tc_mid(agg1, h1p, dinv, b1, w2, blk):
  n, h = h1p.shape
  c = w2.shape[1]

  def body(a_ref, hp_ref, dinv_ref, b_ref, w_ref, h2p_ref):
    s = a_ref[0] + a_ref[1] - hp_ref[...]
    dinv = dinv_ref[...]
    h1 = jnp.maximum(s * dinv + b_ref[...], 0.0)
    h2p_ref[...] = jnp.dot(h1, w_ref[...],
                           preferred_element_type=jnp.float32) * dinv

  grid = (n // blk,)
  return pl.pallas_call(
      body,
      grid=grid,
      in_specs=[
          pl.BlockSpec((NC, blk, h), lambda i: (0, i, 0)),
          pl.BlockSpec((blk, h), lambda i: (i, 0)),
          pl.BlockSpec((blk, 1), lambda i: (i, 0)),
          pl.BlockSpec((1, h), lambda i: (0, 0)),
          pl.BlockSpec((h, c), lambda i: (0, 0)),
      ],
      out_specs=pl.BlockSpec((blk, c), lambda i: (i, 0)),
      out_shape=jax.ShapeDtypeStruct((n, c), jnp.float32),
      compiler_params=_TC_PARAMS,
  )(agg1, h1p, dinv, b1, w2)


def _tc_last(agg2, h2p, dinv, b2, blk):
  n, cp = h2p.shape
  c = b2.shape[1]

  def body(a_ref, hp_ref, dinv_ref, b_ref, fin_ref, logp_ref):
    s = (a_ref[0] + a_ref[1] - hp_ref[...])[:, :c]
    fin = s * dinv_ref[...] + b_ref[...]
    m = jnp.max(fin, axis=1, keepdims=True)
    shifted = fin - m
    lse = jnp.log(jnp.sum(jnp.exp(shifted), axis=1, keepdims=True))
    fin_ref[...] = fin
    logp_ref[...] = shifted - lse

  grid = (n // blk,)
  return pl.pallas_call(
      body,
      grid=grid,
      in_specs=[
          pl.BlockSpec((NC, blk, cp), lambda i: (0, i, 0)),
          pl.BlockSpec((blk, cp), lambda i: (i, 0)),
          pl.BlockSpec((blk, 1), lambda i: (i, 0)),
          pl.BlockSpec((1, c), lambda i: (0, 0)),
      ],
      out_specs=[
          pl.BlockSpec((blk, c), lambda i: (i, 0)),
          pl.BlockSpec((blk, c), lambda i: (i, 0)),
      ],
      out_shape=[
          jax.ShapeDtypeStruct((n, c), jnp.float32),
          jax.ShapeDtypeStruct((n, c), jnp.float32),
      ],
      compiler_params=_TC_PARAMS,
  )(agg2, h2p, dinv, b2)


def kernel(x, edge_index, W1, b1, W2, b2):
  n = x.shape[0]
  src = edge_index[0]
  dst = edge_index[1]
  blk = 1000 if n % 1000 == 0 else 8

  c = W2.shape[1]
  cp = -(-c // 128) * 128
  w2p = jnp.pad(W2, ((0, 0), (0, cp - c))) if cp != c else W2

  deg_parts = _sc_degree(dst, n)
  dinv, h1p = _tc_first(x, W1, deg_parts, blk)
  agg1 = _sc_edge_agg(src, dst, h1p)
  h2p = _tc_mid(agg1, h1p, dinv, b1.reshape(1, -1), w2p, blk)
  agg2 = _sc_edge_agg(src, dst, h2p)
  final, logp = _tc_last(agg2, h2p, dinv, b2.reshape(1, -1), blk)
  return (final, logp)


# speedup vs baseline: 33.3828x; 2.0926x over previous
import functools

import jax
import jax.numpy as jnp
from jax import lax
from jax.experimental import pallas as pl
from jax.experimental.pallas import tpu as pltpu
from jax.experimental.pallas import tpu_sc as plsc

NC = 2
NS = 16
_W = NC * NS
_B = 128
_D = 4
_F = 64


def _mesh():
  return plsc.VectorSubcoreMesh(
      core_axis_name="c", subcore_axis_name="s",
      num_cores=NC, num_subcores=NS)


def _row_split(n_nodes):
  ra = (n_nodes // NS) // 8 * 8
  return ra, n_nodes - NS * ra


def _sc_degree(dst2, dst_t, n_nodes):
  _, nb, _ = dst2.shape
  tail = dst_t.shape[1]
  ra, res = _row_split(n_nodes)

  @functools.partial(
      pl.kernel,
      out_type=jax.ShapeDtypeStruct((NC, n_nodes, 16), jnp.float32),
      mesh=_mesh(),
      scratch_types=[
          pltpu.VMEM((nb, _B), jnp.int32),
          pltpu.VMEM((tail,), jnp.int32),
          pltpu.VMEM((_B, 16), jnp.float32),
          pltpu.VMEM_SHARED((n_nodes, 16), jnp.float32),
          pltpu.SemaphoreType.DMA,
      ],
  )
  def deg_kernel(dst_hbm, dstt_hbm, out_hbm, didx, didx_t, ones_v, cnt_sh,
                 sem):
    cid = lax.axis_index("c")
    sid = lax.axis_index("s")
    wid = sid * NC + cid
    tb = sid * ra

    def fill_row(i, carry):
      ones_v[i, :] = jnp.full((16,), 1.0, jnp.float32)
      return carry
    lax.fori_loop(0, _B, fill_row, 0)
    pltpu.sync_copy(dst_hbm.at[wid], didx)
    pltpu.sync_copy(dstt_hbm.at[wid], didx_t)

    done = 0
    while done < ra:
      sz = min(_B, ra - done)
      pltpu.sync_copy(ones_v.at[pl.ds(0, sz)],
                      cnt_sh.at[pl.ds(tb + done, sz)])
      done += sz
    if res:
      @pl.when(sid == NS - 1)
      def _():
        pltpu.sync_copy(ones_v.at[pl.ds(0, res)],
                        cnt_sh.at[pl.ds(NS * ra, res)])
    plsc.subcore_barrier()

    def batch(i, carry):
      pltpu.async_copy(ones_v, cnt_sh.at[didx.at[i]], sem, add=True)
      @pl.when(i >= _D)
      def _():
        pltpu.make_async_copy(ones_v, cnt_sh.at[didx.at[i]], sem).wait()
      return carry
    lax.fori_loop(0, nb, batch, 0)
    for d in range(min(_D, nb)):
      pltpu.make_async_copy(ones_v, cnt_sh.at[didx.at[d]], sem).wait()
    if tail:
      pltpu.sync_copy(ones_v.at[pl.ds(0, tail)], cnt_sh.at[didx_t], add=True)
    plsc.subcore_barrier()

    pltpu.sync_copy(cnt_sh.at[pl.ds(tb, ra)],
                    out_hbm.at[cid, pl.ds(tb, ra)])
    if res:
      @pl.when(sid == NS - 1)
      def _():
        pltpu.sync_copy(cnt_sh.at[pl.ds(NS * ra, res)],
                        out_hbm.at[cid, pl.ds(NS * ra, res)])

  return deg_kernel(dst2, dst_t)


def _sc_edge_agg(src2, dst2, src_t, dst_t, hp3):
  npass, n_nodes, f = hp3.shape
  _, nb, _ = src2.shape
  tail = src_t.shape[1]
  ra, res = _row_split(n_nodes)
  kmain = nb // _D
  rem = nb % _D

  @functools.partial(
      pl.kernel,
      out_type=jax.ShapeDtypeStruct((NC, npass, n_nodes, f), jnp.float32),
      mesh=_mesh(),
      compiler_params=pltpu.CompilerParams(use_tc_tiling_on_sc=False),
      scratch_types=[
          pltpu.VMEM((nb, _B), jnp.int32),
          pltpu.VMEM((nb, _B), jnp.int32),
          pltpu.VMEM((tail,), jnp.int32),
          pltpu.VMEM((tail,), jnp.int32),
          [pltpu.VMEM((_B, f), jnp.float32)] * _D,
          [pltpu.SemaphoreType.DMA] * _D,
          [pltpu.SemaphoreType.DMA] * _D,
          pltpu.VMEM_SHARED((n_nodes, f), jnp.float32),
      ],
  )
  def agg_kernel(src_hbm, dst_hbm, srct_hbm, dstt_hbm, hp_hbm, out_hbm,
                 sidx, didx, sidx_t, didx_t, rows, gsem, ssem, agg_sh):
    cid = lax.axis_index("c")
    sid = lax.axis_index("s")
    wid = sid * NC + cid
    tb = sid * ra

    pltpu.sync_copy(src_hbm.at[wid], sidx)
    pltpu.sync_copy(dst_hbm.at[wid], didx)
    pltpu.sync_copy(srct_hbm.at[wid], sidx_t)
    pltpu.sync_copy(dstt_hbm.at[wid], didx_t)

    for p in range(npass):
      hview = hp_hbm.at[p]

      pltpu.sync_copy(hview.at[pl.ds(tb, ra)], agg_sh.at[pl.ds(tb, ra)])
      if res:
        @pl.when(sid == NS - 1)
        def _():
          pltpu.sync_copy(hview.at[pl.ds(NS * ra, res)],
                          agg_sh.at[pl.ds(NS * ra, res)])
      plsc.subcore_barrier()

      def start_gather(i, d):
        pltpu.async_copy(hview.at[sidx.at[i]], rows[d], gsem[d])

      def wait_gather(i, d):
        pltpu.make_async_copy(hview.at[sidx.at[i]], rows[d], gsem[d]).wait()

      def start_scatter(i, d):
        pltpu.async_copy(rows[d], agg_sh.at[didx.at[i]], ssem[d], add=True)

      def wait_scatter(i, d):
        pltpu.make_async_copy(rows[d], agg_sh.at[didx.at[i]], ssem[d]).wait()

      for d in range(min(_D, nb)):
        start_gather(d, d)

      def kbody(k, carry):
        for d in range(_D):
          i = k * _D + d
          wait_gather(i, d)
          start_scatter(i, d)
          @pl.when(i + _D < nb)
          def _():
            wait_scatter(i, d)
            start_gather(i + _D, d)
        return carry
      lax.fori_loop(0, kmain, kbody, 0)
      for d in range(rem):
        i = kmain * _D + d
        wait_gather(i, d)
        start_scatter(i, d)
      for d in range(min(_D, nb)):
        wait_scatter(0, d)
      if tail:
        pltpu.async_copy(hview.at[sidx_t], rows[0].at[pl.ds(0, tail)],
                         gsem[0]).wait()
        pltpu.sync_copy(rows[0].at[pl.ds(0, tail)], agg_sh.at[didx_t],
                        add=True)
      plsc.subcore_barrier()

      pltpu.sync_copy(agg_sh.at[pl.ds(tb, ra)],
                      out_hbm.at[cid, p, pl.ds(tb, ra)])
      if res:
        @pl.when(sid == NS - 1)
        def _():
          pltpu.sync_copy(agg_sh.at[pl.ds(NS * ra, res)],
                          out_hbm.at[cid, p, pl.ds(NS * ra, res)])
      if p + 1 < npass:
        plsc.subcore_barrier()

  return agg_kernel(src2, dst2, src_t, dst_t, hp3)


_TC_PARAMS = pltpu.CompilerParams(
    dimension_semantics=("arbitrary",))


def _tc_first(x, w1, deg_parts, blk):
  n, nf = x.shape
  h = w1.shape[1]
  npass = h // _F

  def body(deg_ref, x_ref, w_ref, dinv_ref, h1p_ref):
    d = deg_ref[0, :, 0:1] + deg_ref[1, :, 0:1] - 1.0
    dinv = lax.rsqrt(d)
    dinv_ref[...] = dinv
    r = jnp.dot(x_ref[...], w_ref[...],
                preferred_element_type=jnp.float32) * dinv
    for p in range(npass):
      h1p_ref[p] = r[:, p * _F:(p + 1) * _F]

  grid = (n // blk,)
  return pl.pallas_call(
      body,
      grid=grid,
      in_specs=[
          pl.BlockSpec((NC, blk, 16), lambda i: (0, i, 0)),
          pl.BlockSpec((blk, nf), lambda i: (i, 0)),
          pl.BlockSpec((nf, h), lambda i: (0, 0)),
      ],
      out_specs=[
          pl.BlockSpec((blk, 1), lambda i: (i, 0)),
          pl.BlockSpec((npass, blk, _F), lambda i: (0, i, 0)),
      ],
      out_shape=[
          jax.ShapeDtypeStruct((n, 1), jnp.float32),
          jax.ShapeDtypeStruct((npass, n, _F), jnp.float32),
      ],
      compiler_params=_TC_PARAMS,
  )(deg_parts, x, w1)


def _tc_mid(agg1, h1p, dinv, b1, w2, blk):
  npass, n, _ = h1p.shape
  c = w2.shape[1]

  def body(a_ref, hp_ref, dinv_ref, b_ref, w_ref, h2p_ref):
    s = jnp.concatenate(
        [a_ref[0, p] + a_ref[1, p] - hp_ref[p] for p in range(npass)],
        axis=1)
    dinv = dinv_ref[...]
    h1 = jnp.maximum(s * dinv + b_ref[...], 0.0)
    h2p_ref[...] = jnp.dot(h1, w_ref[...],
                           preferred_element_type=jnp.float32) * dinv

  grid = (n // blk,)
  return pl.pallas_call(
      body,
      grid=grid,
      in_specs=[
          pl.BlockSpec((NC, npass, blk, _F), lambda i: (0, 0, i, 0)),
          pl.BlockSpec((npass, blk, _F), lambda i: (0, i, 0)),
          pl.BlockSpec((blk, 1), lambda i: (i, 0)),
          pl.BlockSpec((1, npass * _F), lambda i: (0, 0)),
          pl.BlockSpec((npass * _F, c), lambda i: (0, 0)),
      ],
      out_specs=pl.BlockSpec((blk, c), lambda i: (i, 0)),
      out_shape=jax.ShapeDtypeStruct((n, c), jnp.float32),
      compiler_params=_TC_PARAMS,
  )(agg1, h1p, dinv, b1, w2)


def _tc_last(agg2, h2p, dinv, b2, blk):
  n, c = h2p.shape

  def body(a_ref, hp_ref, dinv_ref, b_ref, fin_ref, logp_ref):
    s = a_ref[0] + a_ref[1] - hp_ref[...]
    fin = s * dinv_ref[...] + b_ref[...]
    m = jnp.max(fin, axis=1, keepdims=True)
    shifted = fin - m
    lse = jnp.log(jnp.sum(jnp.exp(shifted), axis=1, keepdims=True))
    fin_ref[...] = fin
    logp_ref[...] = shifted - lse

  grid = (n // blk,)
  return pl.pallas_call(
      body,
      grid=grid,
      in_specs=[
          pl.BlockSpec((NC, blk, c), lambda i: (0, i, 0)),
          pl.BlockSpec((blk, c), lambda i: (i, 0)),
          pl.BlockSpec((blk, 1), lambda i: (i, 0)),
          pl.BlockSpec((1, c), lambda i: (0, 0)),
      ],
      out_specs=[
          pl.BlockSpec((blk, c), lambda i: (i, 0)),
          pl.BlockSpec((blk, c), lambda i: (i, 0)),
      ],
      out_shape=[
          jax.ShapeDtypeStruct((n, c), jnp.float32),
          jax.ShapeDtypeStruct((n, c), jnp.float32),
      ],
      compiler_params=_TC_PARAMS,
  )(agg2, h2p, dinv, b2)


def kernel(x, edge_index, W1, b1, W2, b2):
  n = x.shape[0]
  e = edge_index.shape[1]
  src = edge_index[0]
  dst = edge_index[1]
  blk = 1000 if n % 1000 == 0 else 8

  epw = e // _W
  nb = epw // _B
  src_w = src.reshape(_W, epw)
  dst_w = dst.reshape(_W, epw)
  src2 = src_w[:, :nb * _B].reshape(_W, nb, _B)
  dst2 = dst_w[:, :nb * _B].reshape(_W, nb, _B)
  src_t = src_w[:, nb * _B:]
  dst_t = dst_w[:, nb * _B:]

  deg_parts = _sc_degree(dst2, dst_t, n)
  dinv, h1p = _tc_first(x, W1, deg_parts, blk)
  agg1 = _sc_edge_agg(src2, dst2, src_t, dst_t, h1p)
  h2p = _tc_mid(agg1, h1p, dinv, b1.reshape(1, -1), W2, blk)
  c = h2p.shape[1]
  agg2 = _sc_edge_agg(src2, dst2, src_t, dst_t, h2p.reshape(-1, n, _F))
  final, logp = _tc_last(agg2.reshape(NC, n, c), h2p, dinv,
                         b2.reshape(1, -1), blk)
  return (final, logp)


# speedup vs baseline: 33.8370x; 1.0136x over previous
import functools

import jax
import jax.numpy as jnp
from jax import lax
from jax.experimental import pallas as pl
from jax.experimental.pallas import tpu as pltpu
from jax.experimental.pallas import tpu_sc as plsc

NC = 2
NS = 16
_W = NC * NS
_B = 128
_D = 4
_F = 64


def _mesh():
  return plsc.VectorSubcoreMesh(
      core_axis_name="c", subcore_axis_name="s",
      num_cores=NC, num_subcores=NS)


def _row_split(n_nodes):
  ra = (n_nodes // NS) // 8 * 8
  return ra, n_nodes - NS * ra


def _sc_degree(dst2, dst_t, n_nodes):
  _, nb, _ = dst2.shape
  tail = dst_t.shape[1]
  ra, res = _row_split(n_nodes)

  @functools.partial(
      pl.kernel,
      out_type=jax.ShapeDtypeStruct((NC, n_nodes, 16), jnp.float32),
      mesh=_mesh(),
      scratch_types=[
          pltpu.VMEM((nb, _B), jnp.int32),
          pltpu.VMEM((tail,), jnp.int32),
          pltpu.VMEM((_B, 16), jnp.float32),
          pltpu.VMEM_SHARED((n_nodes, 16), jnp.float32),
          pltpu.SemaphoreType.DMA,
      ],
  )
  def deg_kernel(dst_hbm, dstt_hbm, out_hbm, didx, didx_t, ones_v, cnt_sh,
                 sem):
    cid = lax.axis_index("c")
    sid = lax.axis_index("s")
    wid = sid * NC + cid
    tb = sid * ra

    def fill_row(i, carry):
      ones_v[i, :] = jnp.full((16,), 1.0, jnp.float32)
      return carry
    lax.fori_loop(0, _B, fill_row, 0)
    pltpu.sync_copy(dst_hbm.at[wid], didx)
    pltpu.sync_copy(dstt_hbm.at[wid], didx_t)

    done = 0
    while done < ra:
      sz = min(_B, ra - done)
      pltpu.sync_copy(ones_v.at[pl.ds(0, sz)],
                      cnt_sh.at[pl.ds(tb + done, sz)])
      done += sz
    if res:
      @pl.when(sid == NS - 1)
      def _():
        pltpu.sync_copy(ones_v.at[pl.ds(0, res)],
                        cnt_sh.at[pl.ds(NS * ra, res)])
    plsc.subcore_barrier()

    def batch(i, carry):
      pltpu.async_copy(ones_v, cnt_sh.at[didx.at[i]], sem, add=True)
      @pl.when(i >= _D)
      def _():
        pltpu.make_async_copy(ones_v, cnt_sh.at[didx.at[i]], sem).wait()
      return carry
    lax.fori_loop(0, nb, batch, 0)
    for d in range(min(_D, nb)):
      pltpu.make_async_copy(ones_v, cnt_sh.at[didx.at[d]], sem).wait()
    if tail:
      pltpu.sync_copy(ones_v.at[pl.ds(0, tail)], cnt_sh.at[didx_t], add=True)
    plsc.subcore_barrier()

    pltpu.sync_copy(cnt_sh.at[pl.ds(tb, ra)],
                    out_hbm.at[cid, pl.ds(tb, ra)])
    if res:
      @pl.when(sid == NS - 1)
      def _():
        pltpu.sync_copy(cnt_sh.at[pl.ds(NS * ra, res)],
                        out_hbm.at[cid, pl.ds(NS * ra, res)])

  return deg_kernel(dst2, dst_t)


def _sc_edge_agg(src2, dst2, src_t, dst_t, hp3):
  npass, n_nodes, f = hp3.shape
  _, nb, _ = src2.shape
  tail = src_t.shape[1]
  ra, res = _row_split(n_nodes)
  kmain = nb // _D
  rem = nb % _D

  @functools.partial(
      pl.kernel,
      out_type=jax.ShapeDtypeStruct((NC, npass, n_nodes, f), jnp.float32),
      mesh=_mesh(),
      compiler_params=pltpu.CompilerParams(use_tc_tiling_on_sc=False),
      scratch_types=[
          pltpu.VMEM((nb, _B), jnp.int32),
          pltpu.VMEM((nb, _B), jnp.int32),
          pltpu.VMEM((tail,), jnp.int32),
          pltpu.VMEM((tail,), jnp.int32),
          [pltpu.VMEM((_B, f), jnp.float32)] * _D,
          [pltpu.SemaphoreType.DMA] * _D,
          [pltpu.SemaphoreType.DMA] * _D,
          pltpu.VMEM_SHARED((n_nodes, f), jnp.float32),
      ],
  )
  def agg_kernel(src_hbm, dst_hbm, srct_hbm, dstt_hbm, hp_hbm, out_hbm,
                 sidx, didx, sidx_t, didx_t, rows, gsem, ssem, agg_sh):
    cid = lax.axis_index("c")
    sid = lax.axis_index("s")
    wid = sid * NC + cid
    tb = sid * ra

    pltpu.sync_copy(src_hbm.at[wid], sidx)
    pltpu.sync_copy(dst_hbm.at[wid], didx)
    pltpu.sync_copy(srct_hbm.at[wid], sidx_t)
    pltpu.sync_copy(dstt_hbm.at[wid], didx_t)

    for p in range(npass):
      hview = hp_hbm.at[p]

      pltpu.sync_copy(hview.at[pl.ds(tb, ra)], agg_sh.at[pl.ds(tb, ra)])
      if res:
        @pl.when(sid == NS - 1)
        def _():
          pltpu.sync_copy(hview.at[pl.ds(NS * ra, res)],
                          agg_sh.at[pl.ds(NS * ra, res)])
      plsc.subcore_barrier()

      def start_gather(i, d):
        pltpu.async_copy(hview.at[sidx.at[i]], rows[d], gsem[d])

      def wait_gather(i, d):
        pltpu.make_async_copy(hview.at[sidx.at[i]], rows[d], gsem[d]).wait()

      def start_scatter(i, d):
        pltpu.async_copy(rows[d], agg_sh.at[didx.at[i]], ssem[d], add=True)

      def wait_scatter(i, d):
        pltpu.make_async_copy(rows[d], agg_sh.at[didx.at[i]], ssem[d]).wait()

      for d in range(min(_D, nb)):
        start_gather(d, d)

      def kbody(k, carry):
        for d in range(_D):
          i = k * _D + d
          wait_gather(i, d)
          start_scatter(i, d)
          @pl.when(i + _D < nb)
          def _():
            wait_scatter(i, d)
            start_gather(i + _D, d)
        return carry
      lax.fori_loop(0, kmain, kbody, 0)
      for d in range(rem):
        i = kmain * _D + d
        wait_gather(i, d)
        start_scatter(i, d)
      for d in range(min(_D, nb)):
        wait_scatter(0, d)
      if tail:
        pltpu.async_copy(hview.at[sidx_t], rows[0].at[pl.ds(0, tail)],
                         gsem[0]).wait()
        pltpu.sync_copy(rows[0].at[pl.ds(0, tail)], agg_sh.at[didx_t],
                        add=True)
      plsc.subcore_barrier()

      pltpu.sync_copy(agg_sh.at[pl.ds(tb, ra)],
                      out_hbm.at[cid, p, pl.ds(tb, ra)])
      if res:
        @pl.when(sid == NS - 1)
        def _():
          pltpu.sync_copy(agg_sh.at[pl.ds(NS * ra, res)],
                          out_hbm.at[cid, p, pl.ds(NS * ra, res)])
      if p + 1 < npass:
        plsc.subcore_barrier()

  return agg_kernel(src2, dst2, src_t, dst_t, hp3)


_TC_PARAMS = pltpu.CompilerParams(
    dimension_semantics=("arbitrary",))


def _tc_mm(x, w1, blk):
  n, nf = x.shape
  h = w1.shape[1]

  def body(x_ref, w_ref, out_ref):
    out_ref[...] = jnp.dot(x_ref[...], w_ref[...],
                           preferred_element_type=jnp.float32)

  return pl.pallas_call(
      body,
      grid=(n // blk,),
      in_specs=[
          pl.BlockSpec((blk, nf), lambda i: (i, 0)),
          pl.BlockSpec((nf, h), lambda i: (0, 0)),
      ],
      out_specs=pl.BlockSpec((blk, h), lambda i: (i, 0)),
      out_shape=jax.ShapeDtypeStruct((n, h), jnp.float32),
      compiler_params=_TC_PARAMS,
  )(x, w1)


def _tc_scale(h1raw, deg_parts, blk):
  n, h = h1raw.shape
  npass = h // _F

  def body(deg_ref, r_ref, dinv_ref, h1p_ref):
    d = deg_ref[0, :, 0:1] + deg_ref[1, :, 0:1] - 1.0
    dinv = lax.rsqrt(d)
    dinv_ref[...] = dinv
    r = r_ref[...] * dinv
    for p in range(npass):
      h1p_ref[p] = r[:, p * _F:(p + 1) * _F]

  grid = (n // blk,)
  return pl.pallas_call(
      body,
      grid=grid,
      in_specs=[
          pl.BlockSpec((NC, blk, 16), lambda i: (0, i, 0)),
          pl.BlockSpec((blk, h), lambda i: (i, 0)),
      ],
      out_specs=[
          pl.BlockSpec((blk, 1), lambda i: (i, 0)),
          pl.BlockSpec((npass, blk, _F), lambda i: (0, i, 0)),
      ],
      out_shape=[
          jax.ShapeDtypeStruct((n, 1), jnp.float32),
          jax.ShapeDtypeStruct((npass, n, _F), jnp.float32),
      ],
      compiler_params=_TC_PARAMS,
  )(deg_parts, h1raw)


def _tc_mid(agg1, h1p, dinv, b1, w2, blk):
  npass, n, _ = h1p.shape
  c = w2.shape[1]

  def body(a_ref, hp_ref, dinv_ref, b_ref, w_ref, h2p_ref):
    s = jnp.concatenate(
        [a_ref[0, p] + a_ref[1, p] - hp_ref[p] for p in range(npass)],
        axis=1)
    dinv = dinv_ref[...]
    h1 = jnp.maximum(s * dinv + b_ref[...], 0.0)
    h2p_ref[...] = jnp.dot(h1, w_ref[...],
                           preferred_element_type=jnp.float32) * dinv

  grid = (n // blk,)
  return pl.pallas_call(
      body,
      grid=grid,
      in_specs=[
          pl.BlockSpec((NC, npass, blk, _F), lambda i: (0, 0, i, 0)),
          pl.BlockSpec((npass, blk, _F), lambda i: (0, i, 0)),
          pl.BlockSpec((blk, 1), lambda i: (i, 0)),
          pl.BlockSpec((1, npass * _F), lambda i: (0, 0)),
          pl.BlockSpec((npass * _F, c), lambda i: (0, 0)),
      ],
      out_specs=pl.BlockSpec((blk, c), lambda i: (i, 0)),
      out_shape=jax.ShapeDtypeStruct((n, c), jnp.float32),
      compiler_params=_TC_PARAMS,
  )(agg1, h1p, dinv, b1, w2)


def _tc_last(agg2, h2p, dinv, b2, blk):
  n, c = h2p.shape

  def body(a_ref, hp_ref, dinv_ref, b_ref, fin_ref, logp_ref):
    s = a_ref[0] + a_ref[1] - hp_ref[...]
    fin = s * dinv_ref[...] + b_ref[...]
    m = jnp.max(fin, axis=1, keepdims=True)
    shifted = fin - m
    lse = jnp.log(jnp.sum(jnp.exp(shifted), axis=1, keepdims=True))
    fin_ref[...] = fin
    logp_ref[...] = shifted - lse

  grid = (n // blk,)
  return pl.pallas_call(
      body,
      grid=grid,
      in_specs=[
          pl.BlockSpec((NC, blk, c), lambda i: (0, i, 0)),
          pl.BlockSpec((blk, c), lambda i: (i, 0)),
          pl.BlockSpec((blk, 1), lambda i: (i, 0)),
          pl.BlockSpec((1, c), lambda i: (0, 0)),
      ],
      out_specs=[
          pl.BlockSpec((blk, c), lambda i: (i, 0)),
          pl.BlockSpec((blk, c), lambda i: (i, 0)),
      ],
      out_shape=[
          jax.ShapeDtypeStruct((n, c), jnp.float32),
          jax.ShapeDtypeStruct((n, c), jnp.float32),
      ],
      compiler_params=_TC_PARAMS,
  )(agg2, h2p, dinv, b2)


def kernel(x, edge_index, W1, b1, W2, b2):
  n = x.shape[0]
  e = edge_index.shape[1]
  src = edge_index[0]
  dst = edge_index[1]
  blk = 2000 if n % 2000 == 0 else 8

  epw = e // _W
  nb = epw // _B
  src_w = src.reshape(_W, epw)
  dst_w = dst.reshape(_W, epw)
  src2 = src_w[:, :nb * _B].reshape(_W, nb, _B)
  dst2 = dst_w[:, :nb * _B].reshape(_W, nb, _B)
  src_t = src_w[:, nb * _B:]
  dst_t = dst_w[:, nb * _B:]

  deg_parts = _sc_degree(dst2, dst_t, n)
  h1raw = _tc_mm(x, W1, blk)
  dinv, h1p = _tc_scale(h1raw, deg_parts, blk)
  agg1 = _sc_edge_agg(src2, dst2, src_t, dst_t, h1p)
  h2p = _tc_mid(agg1, h1p, dinv, b1.reshape(1, -1), W2, blk)
  c = h2p.shape[1]
  agg2 = _sc_edge_agg(src2, dst2, src_t, dst_t, h2p.reshape(-1, n, _F))
  final, logp = _tc_last(agg2.reshape(NC, n, c), h2p, dinv,
                         b2.reshape(1, -1), blk)
  return (final, logp)
